# baseline (speedup 1.0000x reference)
import jax
import jax.numpy as jnp
from jax import lax
from jax.experimental import pallas as pl
from jax.experimental.pallas import tpu as pltpu
from jax.experimental.pallas import tpu_sc as plsc

BATCH = 4096
CH = 1024
NC = 2
NS = 16
NW = NC * NS
RPW = BATCH // NW
R = 8
NCHUNK = RPW // R
NPAIR = NCHUNK // 2
LANES = 16
KSLICES = CH // LANES


def _permute_body(in_hbm, perm_hbm, out_hbm,
                  perm_v, in0, in1, out0, out1,
                  si0, si1, so0, so1):
    cid = lax.axis_index("c")
    sid = lax.axis_index("s")
    wid = sid * NC + cid
    pltpu.sync_copy(perm_hbm, perm_v)
    row_base = wid * RPW

    ins = (in0, in1)
    outs = (out0, out1)
    sis = (si0, si1)
    sos = (so0, so1)

    rows = [jnp.full((LANES,), r, dtype=jnp.int32) for r in range(R)]

    def start_in(c, p):
        return pltpu.async_copy(
            in_hbm.at[pl.ds(row_base + c * R, R)], ins[p], sis[p])

    def start_out(c, p):
        return pltpu.async_copy(
            outs[p], out_hbm.at[pl.ds(row_base + c * R, R)], sos[p])

    def wait_in(p):
        pltpu.make_async_copy(
            in_hbm.at[pl.ds(row_base, R)], ins[p], sis[p]).wait()

    def wait_out(p):
        pltpu.make_async_copy(
            outs[p], out_hbm.at[pl.ds(row_base, R)], sos[p]).wait()

    def compute(p):
        in_v = ins[p]
        out_v = outs[p]
        prev = None
        for k in range(KSLICES):
            col = k * LANES
            idxv = perm_v[pl.ds(col, LANES)]
            idxv = lax.iota(jnp.int32, LANES) + col
            gs = [plsc.load_gather(in_v, [rows[r], idxv]) for r in range(R)]
            if prev is not None:
                pcol, pgs = prev
                for r in range(R):
                    out_v[r, pl.ds(pcol, LANES)] = pgs[r]
            prev = (col, gs)
        pcol, pgs = prev
        for r in range(R):
            out_v[r, pl.ds(pcol, LANES)] = pgs[r]

    start_in(0, 0)
    start_in(1, 1)

    def pair_body(t, carry):
        for p in (0, 1):
            c = 2 * t + p
            wait_in(p)

            @pl.when(t >= 1)
            def _():
                wait_out(p)

            compute(p)
            start_out(c, p)
            start_in(jnp.minimum(c + 2, NCHUNK - 1), p)
        return carry

    lax.fori_loop(0, NPAIR, pair_body, 0, unroll=False)

    wait_in(0)
    wait_in(1)
    wait_out(0)
    wait_out(1)


@jax.jit
def _permute(x, perm):
    mesh = plsc.VectorSubcoreMesh(core_axis_name="c", subcore_axis_name="s")
    f = pl.kernel(
        _permute_body,
        out_type=jax.ShapeDtypeStruct((BATCH, CH), jnp.float32),
        mesh=mesh,
        scratch_types=[
            pltpu.VMEM((CH,), jnp.int32),
            pltpu.VMEM((R, CH), jnp.float32),
            pltpu.VMEM((R, CH), jnp.float32),
            pltpu.VMEM((R, CH), jnp.float32),
            pltpu.VMEM((R, CH), jnp.float32),
            pltpu.SemaphoreType.DMA,
            pltpu.SemaphoreType.DMA,
            pltpu.SemaphoreType.DMA,
            pltpu.SemaphoreType.DMA,
        ],
        compiler_params=pltpu.CompilerParams(needs_layout_passes=False),
    )
    return f(x, perm)


def kernel(input, perm):
    output = _permute(input, perm)
    log_det = jnp.zeros((), dtype=jnp.float32)
    return (output, log_det)


# speedup vs baseline: 1.4091x; 1.4091x over previous
import jax
import jax.numpy as jnp
from jax import lax
from jax.experimental import pallas as pl
from jax.experimental.pallas import tpu as pltpu
from jax.experimental.pallas import tpu_sc as plsc

BATCH = 4096
CH = 1024
NC = 2
NS = 16
NW = NC * NS
RPW = BATCH // NW
R = 8
NCHUNK = RPW // R
NPAIR = NCHUNK // 2
LANES = 16
KSLICES = CH // LANES


def _permute_body(in_hbm, perm_hbm, out_hbm,
                  perm_v, in0, in1, out0, out1,
                  si0, si1, so0, so1):
    cid = lax.axis_index("c")
    sid = lax.axis_index("s")
    wid = sid * NC + cid
    pltpu.sync_copy(perm_hbm, perm_v)
    row_base = wid * RPW

    ins = (in0, in1)
    outs = (out0, out1)
    sis = (si0, si1)
    sos = (so0, so1)

    rows = [jnp.full((LANES,), r, dtype=jnp.int32) for r in range(R)]

    def start_in(c, p):
        return pltpu.async_copy(
            in_hbm.at[pl.ds(row_base + c * R, R)], ins[p], sis[p])

    def start_out(c, p):
        return pltpu.async_copy(
            outs[p], out_hbm.at[pl.ds(row_base + c * R, R)], sos[p])

    def wait_in(p):
        pltpu.make_async_copy(
            in_hbm.at[pl.ds(row_base, R)], ins[p], sis[p]).wait()

    def wait_out(p):
        pltpu.make_async_copy(
            outs[p], out_hbm.at[pl.ds(row_base, R)], sos[p]).wait()

    def compute(p):
        in_v = ins[p]
        out_v = outs[p]
        prev = None
        for k in range(KSLICES):
            col = k * LANES
            idxv = perm_v[pl.ds(col, LANES)]
            gs = []
            if prev is None:
                for r in range(R):
                    gs.append(plsc.load_gather(in_v, [rows[r], idxv]))
            else:
                pcol, pgs = prev
                for r in range(R):
                    gs.append(plsc.load_gather(in_v, [rows[r], idxv]))
                    out_v[r, pl.ds(pcol, LANES)] = pgs[r]
            prev = (col, gs)
        pcol, pgs = prev
        for r in range(R):
            out_v[r, pl.ds(pcol, LANES)] = pgs[r]

    start_in(0, 0)
    start_in(1, 1)

    def pair_body(t, carry):
        for p in (0, 1):
            c = 2 * t + p
            wait_in(p)

            @pl.when(t >= 1)
            def _():
                wait_out(p)

            compute(p)
            start_out(c, p)
            start_in(jnp.minimum(c + 2, NCHUNK - 1), p)
        return carry

    lax.fori_loop(0, NPAIR, pair_body, 0, unroll=False)

    wait_in(0)
    wait_in(1)
    wait_out(0)
    wait_out(1)


@jax.jit
def _permute(x, perm):
    mesh = plsc.VectorSubcoreMesh(core_axis_name="c", subcore_axis_name="s")
    f = pl.kernel(
        _permute_body,
        out_type=jax.ShapeDtypeStruct((BATCH, CH), jnp.float32),
        mesh=mesh,
        scratch_types=[
            pltpu.VMEM((CH,), jnp.int32),
            pltpu.VMEM((R, CH), jnp.float32),
            pltpu.VMEM((R, CH), jnp.float32),
            pltpu.VMEM((R, CH), jnp.float32),
            pltpu.VMEM((R, CH), jnp.float32),
            pltpu.SemaphoreType.DMA,
            pltpu.SemaphoreType.DMA,
            pltpu.SemaphoreType.DMA,
            pltpu.SemaphoreType.DMA,
        ],
        compiler_params=pltpu.CompilerParams(needs_layout_passes=False),
    )
    return f(x, perm)


def kernel(input, perm):
    output = _permute(input, perm)
    log_det = jnp.zeros((), dtype=jnp.float32)
    return (output, log_det)
